# baseline (speedup 1.0000x reference)
import jax
import jax.numpy as jnp
from jax import lax
from jax.experimental import pallas as pl
from jax.experimental.pallas import tpu as pltpu
from jax.experimental.pallas import tpu_sc as plsc

NC, NS, LANES = 2, 16, 16

GRID = 200
R_GRID = GRID * GRID
R_TOT = R_GRID + NS
ROWS_Z = R_TOT // NS
ROWS_R = R_GRID // NS
CH = 512
CHALF = 32

def _tc_voxel_ids(gt, D, W):
    R3 = gt.shape[0]
    RB = 42
    assert R3 % RB == 0

    def body(g_ref, o_ref):
        g = g_ref[...]
        xs, ys, zs = g[:, 0], g[:, 1], g[:, 2]
        ix = ((xs + 50.0) / 0.5).astype(jnp.int32)
        iy = ((ys + 50.0) / 0.5).astype(jnp.int32)
        iz = ((zs + 10.0) / 20.0).astype(jnp.int32)
        ok = ((ix >= 0) & (ix < GRID) & (iy >= 0) & (iy < GRID)
              & (iz >= 0) & (iz < 1))
        dump = R_GRID + (lax.broadcasted_iota(jnp.int32, ix.shape, 2) & (NS - 1))
        vox = jnp.where(ok, ix * GRID + iy, dump)
        o_ref[...] = jnp.swapaxes(vox, 1, 2)

    return pl.pallas_call(
        body,
        grid=(R3 // RB,),
        in_specs=[pl.BlockSpec((RB, 3, D, W), lambda i: (i, 0, 0, 0))],
        out_specs=pl.BlockSpec((RB, W, D), lambda i: (i, 0, 0)),
        out_shape=jax.ShapeDtypeStruct((R3, W, D), jnp.int32),
    )(gt)


def _sc_scatter(xr, vox):
    Np, C = xr.shape
    nchunk = vox.shape[0]
    assert Np == nchunk * CH and C == 2 * CHALF
    kmax = (nchunk + NS - 1) // NS
    mesh = plsc.VectorSubcoreMesh(
        core_axis_name="c", subcore_axis_name="s",
        num_cores=NC, num_subcores=NS)

    def body(x_hbm, vox_hbm, out_hbm, xbuf, idxbuf, zb, acc):
        core = lax.axis_index("c")
        tid = lax.axis_index("s")
        ch0 = core * CHALF

        def zb_init(i, carry):
            zb[i, pl.ds(0, LANES)] = jnp.zeros((LANES,), jnp.float32)
            zb[i, pl.ds(LANES, LANES)] = jnp.zeros((LANES,), jnp.float32)
            return carry
        lax.fori_loop(0, zb.shape[0], zb_init, 0)

        r0 = tid * ROWS_Z
        off, rem = 0, ROWS_Z
        while rem > 0:
            n = min(rem, zb.shape[0])
            pltpu.sync_copy(zb.at[pl.ds(0, n)], acc.at[pl.ds(r0 + off, n)])
            off += n
            rem -= n
        plsc.subcore_barrier()

        def chunk_body(k, carry):
            c = k * NS + tid

            @pl.when(c < nchunk)
            def _():
                base = c * CH
                pltpu.sync_copy(vox_hbm.at[c], idxbuf)
                pltpu.sync_copy(
                    x_hbm.at[pl.ds(base, CH), pl.ds(ch0, CHALF)], xbuf)
                for j in range(CH // 128):
                    pltpu.sync_copy(xbuf.at[pl.ds(j * 128, 128)],
                                    acc.at[idxbuf.at[j]], add=True)
            return carry
        lax.fori_loop(0, kmax, chunk_body, 0)
        plsc.subcore_barrier()

        rr = tid * ROWS_R
        pltpu.sync_copy(
            acc.at[pl.ds(rr, ROWS_R)],
            out_hbm.at[pl.ds(rr, ROWS_R), pl.ds(ch0, CHALF)])

    f = pl.kernel(
        body,
        out_type=jax.ShapeDtypeStruct((R_GRID, C), jnp.float32),
        mesh=mesh,
        scratch_types=[
            pltpu.VMEM((CH, CHALF), jnp.float32),
            pltpu.VMEM((CH // 128, 128), jnp.int32),
            pltpu.VMEM((512, CHALF), jnp.float32),
            pltpu.VMEM_SHARED((R_TOT, CHALF), jnp.float32),
        ],
        compiler_params=pltpu.CompilerParams(use_tc_tiling_on_sc=False),
    )
    return f(xr, vox)


def _tc_transpose(y):
    B, R, C = y.shape

    def body(in_ref, out_ref):
        out_ref[0] = in_ref[0].T

    return pl.pallas_call(
        body,
        grid=(B,),
        in_specs=[pl.BlockSpec((1, R, C), lambda b: (b, 0, 0))],
        out_specs=pl.BlockSpec((1, C, R), lambda b: (b, 0, 0)),
        out_shape=jax.ShapeDtypeStruct((B, C, R), jnp.float32),
        compiler_params=pltpu.CompilerParams(
            vmem_limit_bytes=100 * 1024 * 1024),
    )(y)


def kernel(x, geometry):
    B, N, D, H, W, C = x.shape
    Np = N * D * H * W
    gt = geometry.transpose(0, 1, 3, 5, 2, 4).reshape(B * N * H, 3, D, W)
    vox = _tc_voxel_ids(gt, D, W).reshape(B, Np // CH, CH // 128, 128)
    zs = []
    for b in range(B):
        xb = x[b].transpose(0, 2, 3, 1, 4).reshape(Np, C)
        yb = _sc_scatter(xb, vox[b])
        zs.append(_tc_transpose(yb[None]))
    return jnp.concatenate(zs, axis=0).reshape(B, C, GRID, GRID)


# speedup vs baseline: 1.4942x; 1.4942x over previous
import jax
import jax.numpy as jnp
from jax import lax
from jax.experimental import pallas as pl
from jax.experimental.pallas import tpu as pltpu
from jax.experimental.pallas import tpu_sc as plsc

NC, NS, LANES = 2, 16, 16

GRID = 200
R_GRID = GRID * GRID
R_TOT = R_GRID + NS
ROWS_Z = R_TOT // NS
ROWS_R = R_GRID // NS
CH = 512
CHALF = 32


def _tc_voxel_ids(gt, D, W):
    R3 = gt.shape[0]
    RB = 42
    assert R3 % RB == 0

    def body(g_ref, o_ref):
        g = g_ref[...]
        xs, ys, zs = g[:, 0], g[:, 1], g[:, 2]
        ix = ((xs + 50.0) / 0.5).astype(jnp.int32)
        iy = ((ys + 50.0) / 0.5).astype(jnp.int32)
        iz = ((zs + 10.0) / 20.0).astype(jnp.int32)
        ok = ((ix >= 0) & (ix < GRID) & (iy >= 0) & (iy < GRID)
              & (iz >= 0) & (iz < 1))
        dump = R_GRID + (lax.broadcasted_iota(jnp.int32, ix.shape, 2) & (NS - 1))
        vox = jnp.where(ok, ix * GRID + iy, dump)
        o_ref[...] = jnp.swapaxes(vox, 1, 2)

    return pl.pallas_call(
        body,
        grid=(R3 // RB,),
        in_specs=[pl.BlockSpec((RB, 3, D, W), lambda i: (i, 0, 0, 0))],
        out_specs=pl.BlockSpec((RB, W, D), lambda i: (i, 0, 0)),
        out_shape=jax.ShapeDtypeStruct((R3, W, D), jnp.int32),
    )(gt)


def _sc_scatter(xr, vox):
    B, Np, C = xr.shape
    nchunk = vox.shape[1]
    assert Np == nchunk * CH and C == 2 * CHALF
    kmax = (nchunk + NS - 1) // NS
    assert kmax % 2 == 0
    mesh = plsc.VectorSubcoreMesh(
        core_axis_name="c", subcore_axis_name="s",
        num_cores=NC, num_subcores=NS)

    def body(x_hbm, vox_hbm, out_hbm, xbuf, idxbuf, zb, acc, semx, semi):
        core = lax.axis_index("c")
        tid = lax.axis_index("s")
        ch0 = core * CHALF

        def zb_init(i, carry):
            zb[i, pl.ds(0, LANES)] = jnp.zeros((LANES,), jnp.float32)
            zb[i, pl.ds(LANES, LANES)] = jnp.zeros((LANES,), jnp.float32)
            return carry
        lax.fori_loop(0, zb.shape[0], zb_init, 0)

        def in_copies(b, k, slot):
            c = k * NS + tid
            x_cp = pltpu.make_async_copy(
                x_hbm.at[b, pl.ds(c * CH, CH), pl.ds(ch0, CHALF)],
                xbuf.at[slot], semx.at[slot])
            i_cp = pltpu.make_async_copy(
                vox_hbm.at[b, c], idxbuf.at[slot], semi.at[slot])
            return c, x_cp, i_cp

        def start_in(b, k, slot):
            c, x_cp, i_cp = in_copies(b, k, slot)

            @pl.when(c < nchunk)
            def _():
                x_cp.start()
                i_cp.start()

        def use_chunk(b, k, slot):
            c, x_cp, i_cp = in_copies(b, k, slot)

            @pl.when(c < nchunk)
            def _():
                x_cp.wait()
                i_cp.wait()
                for j in range(CH // 128):
                    pltpu.sync_copy(xbuf.at[slot, pl.ds(j * 128, 128)],
                                    acc.at[idxbuf.at[slot, j]], add=True)

        for b in range(B):
            r0 = tid * ROWS_Z
            off, rem = 0, ROWS_Z
            while rem > 0:
                n = min(rem, zb.shape[0])
                pltpu.sync_copy(zb.at[pl.ds(0, n)], acc.at[pl.ds(r0 + off, n)])
                off += n
                rem -= n
            plsc.subcore_barrier()

            start_in(b, 0, 0)

            def pair_body(i, carry):
                k = i * 2
                start_in(b, k + 1, 1)
                use_chunk(b, k, 0)
                start_in(b, k + 2, 0)
                use_chunk(b, k + 1, 1)
                return carry
            lax.fori_loop(0, kmax // 2 - 1, pair_body, 0)

            k = kmax - 2
            start_in(b, k + 1, 1)
            use_chunk(b, k, 0)
            use_chunk(b, k + 1, 1)
            plsc.subcore_barrier()

            rr = tid * ROWS_R
            pltpu.sync_copy(
                acc.at[pl.ds(rr, ROWS_R)],
                out_hbm.at[b, pl.ds(rr, ROWS_R), pl.ds(ch0, CHALF)])
            plsc.subcore_barrier()

    f = pl.kernel(
        body,
        out_type=jax.ShapeDtypeStruct((B, R_GRID, C), jnp.float32),
        mesh=mesh,
        scratch_types=[
            pltpu.VMEM((2, CH, CHALF), jnp.float32),
            pltpu.VMEM((2, CH // 128, 128), jnp.int32),
            pltpu.VMEM((512, CHALF), jnp.float32),
            pltpu.VMEM_SHARED((R_TOT, CHALF), jnp.float32),
            pltpu.SemaphoreType.DMA((2,)),
            pltpu.SemaphoreType.DMA((2,)),
        ],
        compiler_params=pltpu.CompilerParams(use_tc_tiling_on_sc=False),
    )
    return f(xr, vox)


def _tc_transpose(y):
    B, R, C = y.shape

    def body(in_ref, out_ref):
        out_ref[0] = in_ref[0].T

    return pl.pallas_call(
        body,
        grid=(B,),
        in_specs=[pl.BlockSpec((1, R, C), lambda b: (b, 0, 0))],
        out_specs=pl.BlockSpec((1, C, R), lambda b: (b, 0, 0)),
        out_shape=jax.ShapeDtypeStruct((B, C, R), jnp.float32),
        compiler_params=pltpu.CompilerParams(
            vmem_limit_bytes=100 * 1024 * 1024),
    )(y)


def kernel(x, geometry):
    B, N, D, H, W, C = x.shape
    Np = N * D * H * W
    xr = x.transpose(0, 1, 3, 4, 2, 5).reshape(B, Np, C)
    gt = geometry.transpose(0, 1, 3, 5, 2, 4).reshape(B * N * H, 3, D, W)
    vox = _tc_voxel_ids(gt, D, W).reshape(B, Np // CH, CH // 128, 128)
    y = _sc_scatter(xr, vox)
    z = _tc_transpose(y)
    return z.reshape(B, C, GRID, GRID)
